# Initial kernel scaffold; baseline (speedup 1.0000x reference)
import jax
import jax.numpy as jnp
from jax.experimental import pallas as pl


def kernel(x, edge_index, W1, a_src1, a_dst1, b1, W2, a_src2, a_dst2, b2):
    raise NotImplementedError("write your pallas kernel here")


# speedup vs baseline: 23.0476x; 23.0476x over previous
import functools

import jax
import jax.numpy as jnp
from jax import lax
from jax.experimental import pallas as pl
from jax.experimental.pallas import tpu as pltpu
from jax.experimental.pallas import tpu_sc as plsc

N = 10000
D = 128
E = 320000
NEG = 0.2

NC = 2
NS = 16
NW = NC * NS
EPW = E // NW
K = 80
NCHUNK = EPW // K
NP = 10240
RPT = NP // NS

BN = 2000


def _attn_tail(h, asrc, adst):
    p = jnp.sum(h * asrc, axis=1)
    q = jnp.sum(h * adst, axis=1)
    t = p + q
    w0 = jnp.exp(jnp.where(t >= 0.0, t, NEG * t))
    return p, q, w0


def _pre_body(x_ref, w_ref, asrc_ref, adst_ref,
              h_ref, p_ref, q_ref, n0_ref, d0_ref):
    h = jnp.dot(x_ref[...], w_ref[...], preferred_element_type=jnp.float32)
    p, q, w0 = _attn_tail(h, asrc_ref[...], adst_ref[...])
    h_ref[...] = h
    p_ref[...] = p
    q_ref[...] = q
    n0_ref[...] = h * w0[:, None]
    d0_ref[...] = w0


def _mid_body(n0_ref, pa_ref, pb_ref, d0_ref, da_ref, db_ref, b_ref,
              w_ref, asrc_ref, adst_ref,
              h_ref, p_ref, q_ref, n0o_ref, d0o_ref):
    den = d0_ref[...] + da_ref[...] + db_ref[...] + 1e-16
    xr = (n0_ref[...] + pa_ref[...] + pb_ref[...]) / den[:, None] + b_ref[...]
    xr = jnp.maximum(xr, 0.0)
    h = jnp.dot(xr, w_ref[...], preferred_element_type=jnp.float32)
    p, q, w0 = _attn_tail(h, asrc_ref[...], adst_ref[...])
    h_ref[...] = h
    p_ref[...] = p
    q_ref[...] = q
    n0o_ref[...] = h * w0[:, None]
    d0o_ref[...] = w0


def _fin_body(n0_ref, pa_ref, pb_ref, d0_ref, da_ref, db_ref, b_ref, out_ref):
    den = d0_ref[...] + da_ref[...] + db_ref[...] + 1e-16
    out_ref[...] = (n0_ref[...] + pa_ref[...] + pb_ref[...]) / den[:, None] \
        + b_ref[...]


_OUT_MAT = jax.ShapeDtypeStruct((N, D), jnp.float32)
_OUT_VEC = jax.ShapeDtypeStruct((N,), jnp.float32)


def _tc_pre(x, W, a_src, a_dst):
    return pl.pallas_call(
        _pre_body,
        out_shape=[_OUT_MAT, _OUT_VEC, _OUT_VEC, _OUT_MAT, _OUT_VEC],
    )(x, W, a_src.reshape(1, D), a_dst.reshape(1, D))


def _tc_mid(n0, pa, pb, d0, da, db, b, W, a_src, a_dst):
    return pl.pallas_call(
        _mid_body,
        out_shape=[_OUT_MAT, _OUT_VEC, _OUT_VEC, _OUT_MAT, _OUT_VEC],
    )(n0, pa, pb, d0, da, db, b.reshape(1, D), W,
      a_src.reshape(1, D), a_dst.reshape(1, D))


def _tc_fin(n0, pa, pb, d0, da, db, b):
    return pl.pallas_call(
        _fin_body,
        out_shape=_OUT_MAT,
    )(n0, pa, pb, d0, da, db, b.reshape(1, D))


_MESH = plsc.VectorSubcoreMesh(
    core_axis_name="c", subcore_axis_name="s", num_cores=NC, num_subcores=NS)


@functools.partial(
    pl.kernel,
    out_type=[jax.ShapeDtypeStruct((NC * NP, D), jnp.float32),
              jax.ShapeDtypeStruct((NC * NP,), jnp.float32)],
    mesh=_MESH,
    compiler_params=pltpu.CompilerParams(needs_layout_passes=False),
    scratch_types=[
        pltpu.VMEM((N,), jnp.float32),
        pltpu.VMEM((N,), jnp.float32),
        pltpu.VMEM((K,), jnp.int32),
        pltpu.VMEM((K,), jnp.int32),
        pltpu.VMEM((K,), jnp.float32),
        pltpu.VMEM((K, D), jnp.float32),
        pltpu.VMEM_SHARED((NP, D), jnp.float32),
        pltpu.VMEM_SHARED((NP,), jnp.float32),
    ],
)
def _sc_edge(src_hbm, dst_hbm, h_hbm, p_hbm, q_hbm, z_hbm, zn_hbm,
             parts_hbm, dparts_hbm,
             p_loc, q_loc, src_c, dst_c, w_c, rows, numer_sh, den_sh):
    c = lax.axis_index("c")
    s = lax.axis_index("s")

    pltpu.sync_copy(z_hbm.at[pl.ds(s * RPT, RPT)],
                    numer_sh.at[pl.ds(s * RPT, RPT)])
    @pl.when(s == 0)
    def _():
        pltpu.sync_copy(zn_hbm, den_sh)
    pltpu.sync_copy(p_hbm, p_loc)
    pltpu.sync_copy(q_hbm, q_loc)
    plsc.subcore_barrier()

    ebase = (c * NS + s) * EPW

    def chunk_body(ci, carry):
        off = ebase + ci * K
        pltpu.sync_copy(src_hbm.at[pl.ds(off, K)], src_c)
        pltpu.sync_copy(dst_hbm.at[pl.ds(off, K)], dst_c)
        pltpu.sync_copy(h_hbm.at[src_c], rows)

        def w_body(j, carry2):
            sv = src_c[pl.ds(j * 16, 16)]
            dv = dst_c[pl.ds(j * 16, 16)]
            t = plsc.load_gather(p_loc, [sv]) + plsc.load_gather(q_loc, [dv])
            t = jnp.where(t >= 0.0, t, NEG * t)
            w_c[pl.ds(j * 16, 16)] = jnp.exp(t)
            return carry2
        lax.fori_loop(0, K // 16, w_body, 0)

        def r_body(e, carry2):
            wb = plsc.load_gather(w_c, [jnp.full((16,), e, jnp.int32)])
            for j in range(D // 16):
                rows[e, pl.ds(j * 16, 16)] = rows[e, pl.ds(j * 16, 16)] * wb
            return carry2
        lax.fori_loop(0, K, r_body, 0)

        pltpu.sync_copy(rows, numer_sh.at[dst_c], add=True)
        pltpu.sync_copy(w_c, den_sh.at[dst_c], add=True)
        return carry
    lax.fori_loop(0, NCHUNK, chunk_body, 0)

    plsc.subcore_barrier()
    pltpu.sync_copy(numer_sh.at[pl.ds(s * RPT, RPT)],
                    parts_hbm.at[pl.ds(c * NP + s * RPT, RPT)])
    @pl.when(s == 0)
    def _():
        pltpu.sync_copy(den_sh, dparts_hbm.at[pl.ds(c * NP, NP)])


def kernel(x, edge_index, W1, a_src1, a_dst1, b1, W2, a_src2, a_dst2, b2):
    src = edge_index[0]
    dst = edge_index[1]
    zrow = jnp.zeros((NP, D), jnp.float32)
    zn = jnp.zeros((NP,), jnp.float32)

    h1, p1, q1, n01, d01 = _tc_pre(x, W1, a_src1, a_dst1)
    parts1, dparts1 = _sc_edge(src, dst, h1, p1, q1, zrow, zn)
    h2, p2, q2, n02, d02 = _tc_mid(
        n01, parts1[:N], parts1[NP:NP + N], d01, dparts1[:N],
        dparts1[NP:NP + N], b1, W2, a_src2, a_dst2)
    parts2, dparts2 = _sc_edge(src, dst, h2, p2, q2, zrow, zn)
    out = _tc_fin(n02, parts2[:N], parts2[NP:NP + N], d02, dparts2[:N],
                  dparts2[NP:NP + N], b2)
    return out
